# baseline (speedup 1.0000x reference)
import functools

import jax
import jax.numpy as jnp
from jax import lax
from jax.experimental import pallas as pl
from jax.experimental.pallas import tpu as pltpu
from jax.experimental.pallas import tpu_sc as plsc

VOCAB = 1000
D = 1000
BATCH = 4096
SEQ = 20
B = BATCH * SEQ

NC, NS = 2, 16
NW = NC * NS
KB = 4
C = KB * SEQ
BATCH_PER_W = BATCH // NW
N_CHUNKS = BATCH_PER_W // KB


def _body(idx_hbm, tab_hbm, out_hbm, idx_v, rows_v, gsem, ssem):
    wid = lax.axis_index("s") * NC + lax.axis_index("c")
    bbase = wid * BATCH_PER_W
    pltpu.sync_copy(idx_hbm.at[wid], idx_v)

    def chunk(j, carry):
        pltpu.async_copy(tab_hbm.at[idx_v.at[j]], rows_v, gsem).wait()
        for r in range(KB):
            pltpu.async_copy(
                rows_v.at[pl.ds(r * SEQ, SEQ)],
                out_hbm.at[bbase + j * KB + r],
                ssem,
            ).wait()
        return carry

    lax.fori_loop(0, N_CHUNKS, chunk, 0)


@functools.partial(jax.jit, static_argnums=())
def _gather_rows(idx, table):
    k = pl.kernel(
        _body,
        out_type=jax.ShapeDtypeStruct((BATCH, SEQ, D), jnp.float32),
        mesh=plsc.VectorSubcoreMesh(core_axis_name="c", subcore_axis_name="s"),
        scratch_types=[
            pltpu.VMEM((N_CHUNKS, C), jnp.int32),
            pltpu.VMEM((C, D), jnp.float32),
            pltpu.SemaphoreType.DMA,
            pltpu.SemaphoreType.DMA,
        ],
        compiler_params=pltpu.CompilerParams(use_tc_tiling_on_sc=False),
    )
    return k(idx, table)


def kernel(inputs, embedding_table):
    idx = inputs.reshape(NW, N_CHUNKS, C)
    return _gather_rows(idx, embedding_table)
